# baseline (speedup 1.0000x reference)
import functools

import jax
import jax.numpy as jnp
from jax import lax
from jax.experimental import pallas as pl
from jax.experimental.pallas import tpu as pltpu
from jax.experimental.pallas import tpu_sc as plsc

N = 10000
E = 320000
D = 128
H = 64
G = 32
C = 10

NC = 2
NS = 16
NW = NC * NS
CHUNK = 80
NCHUNKS = 128
EDGES_PER_W = NCHUNKS * CHUNK
E_PAD = NW * EDGES_PER_W - E
ROWS_A = 624
ROWS_REM = N - NS * ROWS_A

ROW_BLK = 2000
GRID_N = N // ROW_BLK

_BN_SCALE = 1.0 / (1.0 + 1e-5) ** 0.5


def _elu(t):
    return jnp.where(t > 0, t, jnp.exp(jnp.minimum(t, 0.0)) - 1.0)


def _agg_body(y_hbm, srcs_hbm, dsts_hbm, out_hbm, src_v, dst_v, buf0, buf1,
              buf2, buf3, acc_sh, gsem, ssem):
    c = lax.axis_index("c")
    s = lax.axis_index("s")
    wid = c * NS + s
    pltpu.sync_copy(y_hbm.at[pl.ds(s * ROWS_A, ROWS_A)],
                    acc_sh.at[pl.ds(s * ROWS_A, ROWS_A)])

    @pl.when(s == NS - 1)
    def _():
        pltpu.sync_copy(y_hbm.at[pl.ds(NS * ROWS_A, ROWS_REM)],
                        acc_sh.at[pl.ds(NS * ROWS_A, ROWS_REM)])
    pltpu.sync_copy(srcs_hbm.at[wid], src_v)
    pltpu.sync_copy(dsts_hbm.at[wid], dst_v)
    plsc.subcore_barrier()

    bufs = (buf0, buf1, buf2, buf3)

    def g_fire(j, buf):
        pltpu.async_copy(y_hbm.at[src_v.at[j]], buf, gsem)

    def g_wait(j, buf):
        pltpu.make_async_copy(y_hbm.at[src_v.at[j]], buf, gsem).wait()

    def s_fire(j, buf):
        pltpu.async_copy(buf, acc_sh.at[dst_v.at[j]], ssem, add=True)

    def s_wait(j, buf):
        pltpu.make_async_copy(buf, acc_sh.at[dst_v.at[j]], ssem).wait()

    for b in range(4):
        g_fire(b, bufs[b])

    def body(t, carry):
        j = 4 * t
        for b in range(4):
            g_wait(j + b, bufs[b])
            s_fire(j + b, bufs[b])

        @pl.when(t < NCHUNKS // 4 - 1)
        def _():
            for b in range(4):
                s_wait(j + b, bufs[b])
                g_fire(j + 4 + b, bufs[b])
        return carry

    lax.fori_loop(0, NCHUNKS // 4, body, 0)
    for b in range(4):
        s_wait(NCHUNKS - 4 + b, bufs[b])
    plsc.subcore_barrier()
    pltpu.sync_copy(acc_sh.at[pl.ds(s * ROWS_A, ROWS_A)],
                    out_hbm.at[c, pl.ds(s * ROWS_A, ROWS_A)])

    @pl.when(s == NS - 1)
    def _():
        pltpu.sync_copy(acc_sh.at[pl.ds(NS * ROWS_A, ROWS_REM)],
                        out_hbm.at[c, pl.ds(NS * ROWS_A, ROWS_REM)])


def _make_agg():
    mesh = plsc.VectorSubcoreMesh(core_axis_name="c", subcore_axis_name="s",
                                  num_cores=NC, num_subcores=NS)
    return pl.kernel(
        _agg_body,
        out_type=jax.ShapeDtypeStruct((NC, N, H), jnp.float32),
        mesh=mesh,
        scratch_types=[
            pltpu.VMEM((NCHUNKS, CHUNK), jnp.int32),
            pltpu.VMEM((NCHUNKS, CHUNK), jnp.int32),
            pltpu.VMEM((CHUNK, H), jnp.float32),
            pltpu.VMEM((CHUNK, H), jnp.float32),
            pltpu.VMEM((CHUNK, H), jnp.float32),
            pltpu.VMEM((CHUNK, H), jnp.float32),
            pltpu.VMEM_SHARED((N + 128, H), jnp.float32),
            pltpu.SemaphoreType.DMA,
            pltpu.SemaphoreType.DMA,
        ],
        compiler_params=pltpu.CompilerParams(use_tc_tiling_on_sc=False),
    )


_sc_aggregate = _make_agg()


def _head_body(x_ref, w_ref, o_ref):
    o_ref[...] = jnp.dot(x_ref[...], w_ref[...],
                         preferred_element_type=jnp.float32)


def _head_matmul(x, w):
    d_in = x.shape[1]
    return pl.pallas_call(
        _head_body,
        grid=(GRID_N,),
        in_specs=[
            pl.BlockSpec((ROW_BLK, d_in), lambda i: (i, 0)),
            pl.BlockSpec((d_in, H), lambda i: (0, 0)),
        ],
        out_specs=pl.BlockSpec((ROW_BLK, H), lambda i: (i, 0)),
        out_shape=jax.ShapeDtypeStruct((N, H), jnp.float32),
    )(x, w)


def _tail_head_body(p0_ref, p1_ref, y_ref, b1_ref, g_ref, bt_ref, w2_ref,
                    b2_ref, w1n_ref, o_ref, *, use_bn):
    t = p0_ref[0] + p1_ref[0] - y_ref[...] + b1_ref[...]
    if use_bn:
        t = t * (_BN_SCALE * g_ref[...]) + bt_ref[...]
    t = _elu(t)
    t = jnp.dot(t, w2_ref[...], preferred_element_type=jnp.float32)
    z = _elu(t + b2_ref[...])
    o_ref[...] = jnp.dot(z, w1n_ref[...], preferred_element_type=jnp.float32)


def _tail_head(p, y, b1, gamma, beta, w2, b2, w1_next, use_bn):
    row = lambda v: v.reshape(1, H)
    vec_spec = pl.BlockSpec((1, H), lambda i: (0, 0))
    mat_spec = pl.BlockSpec((H, H), lambda i: (0, 0))
    blk_spec = pl.BlockSpec((ROW_BLK, H), lambda i: (i, 0))
    p_spec0 = pl.BlockSpec((1, ROW_BLK, H), lambda i: (0, i, 0))
    p_spec1 = pl.BlockSpec((1, ROW_BLK, H), lambda i: (1, i, 0))
    return pl.pallas_call(
        functools.partial(_tail_head_body, use_bn=use_bn),
        grid=(GRID_N,),
        in_specs=[p_spec0, p_spec1, blk_spec, vec_spec, vec_spec, vec_spec,
                  mat_spec, vec_spec, mat_spec],
        out_specs=blk_spec,
        out_shape=jax.ShapeDtypeStruct((N, H), jnp.float32),
    )(p, p, y, row(b1), row(gamma), row(beta), w2, row(b2), w1_next)


def _final_body(p0_ref, p1_ref, y_ref, b1_ref, w2_ref, b2_ref, batch_ref,
                mw1_ref, mb1_ref, mw2_ref, mb2_ref, mw3_ref, mb3_ref,
                o_ref, acc_ref):
    i = pl.program_id(0)
    t = p0_ref[0] + p1_ref[0] - y_ref[...] + b1_ref[...]
    t = _elu(t)
    t = jnp.dot(t, w2_ref[...], preferred_element_type=jnp.float32)
    z = _elu(t + b2_ref[...])
    b = batch_ref[0, 0, :]
    onehot_t = (lax.broadcasted_iota(jnp.int32, (G, ROW_BLK), 0)
                == b[None, :]).astype(jnp.float32)
    part = jnp.dot(onehot_t, z, preferred_element_type=jnp.float32)

    @pl.when(i == 0)
    def _():
        acc_ref[...] = jnp.zeros_like(acc_ref)

    acc_ref[...] += part

    @pl.when(i == GRID_N - 1)
    def _():
        pooled = acc_ref[...]
        r = _elu(jnp.dot(pooled, mw1_ref[...],
                         preferred_element_type=jnp.float32) + mb1_ref[...])
        r = _elu(jnp.dot(r, mw2_ref[...],
                         preferred_element_type=jnp.float32) + mb2_ref[...])
        o_ref[...] = jnp.dot(r, mw3_ref[...],
                             preferred_element_type=jnp.float32) + mb3_ref[...]


def _final(p, y, b1, w2, b2, batch3, mw1, mb1, mw2, mb2, mw3, mb3):
    vec = lambda v: v.reshape(1, -1)
    vec_spec = lambda n: pl.BlockSpec((1, n), lambda i: (0, 0))
    mat_spec = lambda m, n: pl.BlockSpec((m, n), lambda i: (0, 0))
    blk_spec = pl.BlockSpec((ROW_BLK, H), lambda i: (i, 0))
    return pl.pallas_call(
        _final_body,
        grid=(GRID_N,),
        in_specs=[
            pl.BlockSpec((1, ROW_BLK, H), lambda i: (0, i, 0)),
            pl.BlockSpec((1, ROW_BLK, H), lambda i: (1, i, 0)),
            blk_spec,
            vec_spec(H), mat_spec(H, H), vec_spec(H),
            pl.BlockSpec((1, 1, ROW_BLK), lambda i: (i, 0, 0)),
            mat_spec(H, H), vec_spec(H),
            mat_spec(H, H // 2), vec_spec(H // 2),
            mat_spec(H // 2, C), vec_spec(C),
        ],
        out_specs=pl.BlockSpec((G, C), lambda i: (0, 0)),
        out_shape=jax.ShapeDtypeStruct((G, C), jnp.float32),
        scratch_shapes=[pltpu.VMEM((G, H), jnp.float32)],
    )(p, p, y, vec(b1), w2, vec(b2), batch3, mw1, vec(mb1), mw2, vec(mb2),
      mw3, vec(mb3))


def kernel(x, edge_index, batch,
           pre0_w1, pre0_b1, pre0_gamma, pre0_beta, pre0_w2, pre0_b2,
           pre1_w1, pre1_b1, pre1_gamma, pre1_beta, pre1_w2, pre1_b2,
           post0_w1, post0_b1, post0_w2, post0_b2,
           mlp_w1, mlp_b1, mlp_w2, mlp_b2, mlp_w3, mlp_b3):
    pad_src = jnp.zeros((E_PAD,), jnp.int32)
    pad_dst = N + (jnp.arange(E_PAD, dtype=jnp.int32) % 128)
    srcs = jnp.concatenate([edge_index[0], pad_src]).reshape(NW, NCHUNKS, CHUNK)
    dsts = jnp.concatenate([edge_index[1], pad_dst]).reshape(NW, NCHUNKS, CHUNK)
    batch3 = batch.reshape(GRID_N, 1, ROW_BLK)

    y0 = _head_matmul(x, pre0_w1)
    p0 = _sc_aggregate(y0, srcs, dsts)
    y1 = _tail_head(p0, y0, pre0_b1, pre0_gamma, pre0_beta, pre0_w2, pre0_b2,
                    pre1_w1, use_bn=True)
    p1 = _sc_aggregate(y1, srcs, dsts)
    y2 = _tail_head(p1, y1, pre1_b1, pre1_gamma, pre1_beta, pre1_w2, pre1_b2,
                    post0_w1, use_bn=True)
    p2 = _sc_aggregate(y2, srcs, dsts)
    out = _final(p2, y2, post0_b1, post0_w2, post0_b2, batch3,
                 mlp_w1, mlp_b1, mlp_w2, mlp_b2, mlp_w3, mlp_b3)
    return (out, edge_index, batch)


# speedup vs baseline: 1.6386x; 1.6386x over previous
import functools

import jax
import jax.numpy as jnp
from jax import lax
from jax.experimental import pallas as pl
from jax.experimental.pallas import tpu as pltpu
from jax.experimental.pallas import tpu_sc as plsc

N = 10000
E = 320000
D = 128
H = 64
G = 32
C = 10

NC = 2
NS = 16
NW = NC * NS
CHUNK = 112
NCHUNKS = 90
EDGES_PER_W = NCHUNKS * CHUNK
E_PAD = NW * EDGES_PER_W - E
ROWS_A = 624
ROWS_REM = N - NS * ROWS_A

ROW_BLK = 2000
GRID_N = N // ROW_BLK

_BN_SCALE = 1.0 / (1.0 + 1e-5) ** 0.5


def _elu(t):
    return jnp.where(t > 0, t, jnp.exp(jnp.minimum(t, 0.0)) - 1.0)


def _agg_body(y_hbm, srcs_hbm, dsts_hbm, out_hbm, src_v, dst_v, buf0, buf1,
              acc_sh, gsem, ssem):
    c = lax.axis_index("c")
    s = lax.axis_index("s")
    wid = c * NS + s
    pltpu.sync_copy(y_hbm.at[pl.ds(s * ROWS_A, ROWS_A)],
                    acc_sh.at[pl.ds(s * ROWS_A, ROWS_A)])

    @pl.when(s == NS - 1)
    def _():
        pltpu.sync_copy(y_hbm.at[pl.ds(NS * ROWS_A, ROWS_REM)],
                        acc_sh.at[pl.ds(NS * ROWS_A, ROWS_REM)])
    pltpu.sync_copy(srcs_hbm.at[wid], src_v)
    pltpu.sync_copy(dsts_hbm.at[wid], dst_v)
    plsc.subcore_barrier()

    def g_fire(j, buf):
        pltpu.async_copy(y_hbm.at[src_v.at[j]], buf, gsem)

    def g_wait(j, buf):
        pltpu.make_async_copy(y_hbm.at[src_v.at[j]], buf, gsem).wait()

    def s_sync(j, buf):
        pltpu.sync_copy(buf, acc_sh.at[dst_v.at[j]], add=True)

    g_fire(0, buf0)

    def body(t, carry):
        j0 = 2 * t
        j1 = j0 + 1
        g_fire(j1, buf1)
        g_wait(j0, buf0)
        s_sync(j0, buf0)

        @pl.when(t < NCHUNKS // 2 - 1)
        def _():
            g_fire(j0 + 2, buf0)
        g_wait(j1, buf1)
        s_sync(j1, buf1)
        return carry

    lax.fori_loop(0, NCHUNKS // 2, body, 0)
    plsc.subcore_barrier()
    pltpu.sync_copy(acc_sh.at[pl.ds(s * ROWS_A, ROWS_A)],
                    out_hbm.at[c, pl.ds(s * ROWS_A, ROWS_A)])

    @pl.when(s == NS - 1)
    def _():
        pltpu.sync_copy(acc_sh.at[pl.ds(NS * ROWS_A, ROWS_REM)],
                        out_hbm.at[c, pl.ds(NS * ROWS_A, ROWS_REM)])


def _make_agg():
    mesh = plsc.VectorSubcoreMesh(core_axis_name="c", subcore_axis_name="s",
                                  num_cores=NC, num_subcores=NS)
    return pl.kernel(
        _agg_body,
        out_type=jax.ShapeDtypeStruct((NC, N, H), jnp.float32),
        mesh=mesh,
        scratch_types=[
            pltpu.VMEM((NCHUNKS, CHUNK), jnp.int32),
            pltpu.VMEM((NCHUNKS, CHUNK), jnp.int32),
            pltpu.VMEM((CHUNK, H), jnp.float32),
            pltpu.VMEM((CHUNK, H), jnp.float32),
            pltpu.VMEM_SHARED((N + 128, H), jnp.float32),
            pltpu.SemaphoreType.DMA,
            pltpu.SemaphoreType.DMA,
        ],
        compiler_params=pltpu.CompilerParams(use_tc_tiling_on_sc=False),
    )


_sc_aggregate = _make_agg()


def _head_body(x_ref, w_ref, o_ref):
    o_ref[...] = jnp.dot(x_ref[...], w_ref[...],
                         preferred_element_type=jnp.float32)


def _head_matmul(x, w):
    d_in = x.shape[1]
    return pl.pallas_call(
        _head_body,
        grid=(GRID_N,),
        in_specs=[
            pl.BlockSpec((ROW_BLK, d_in), lambda i: (i, 0)),
            pl.BlockSpec((d_in, H), lambda i: (0, 0)),
        ],
        out_specs=pl.BlockSpec((ROW_BLK, H), lambda i: (i, 0)),
        out_shape=jax.ShapeDtypeStruct((N, H), jnp.float32),
    )(x, w)


def _tail_head_body(p0_ref, p1_ref, y_ref, b1_ref, g_ref, bt_ref, w2_ref,
                    b2_ref, w1n_ref, o_ref, *, use_bn):
    t = p0_ref[0] + p1_ref[0] - y_ref[...] + b1_ref[...]
    if use_bn:
        t = t * (_BN_SCALE * g_ref[...]) + bt_ref[...]
    t = _elu(t)
    t = jnp.dot(t, w2_ref[...], preferred_element_type=jnp.float32)
    z = _elu(t + b2_ref[...])
    o_ref[...] = jnp.dot(z, w1n_ref[...], preferred_element_type=jnp.float32)


def _tail_head(p, y, b1, gamma, beta, w2, b2, w1_next, use_bn):
    row = lambda v: v.reshape(1, H)
    vec_spec = pl.BlockSpec((1, H), lambda i: (0, 0))
    mat_spec = pl.BlockSpec((H, H), lambda i: (0, 0))
    blk_spec = pl.BlockSpec((ROW_BLK, H), lambda i: (i, 0))
    p_spec0 = pl.BlockSpec((1, ROW_BLK, H), lambda i: (0, i, 0))
    p_spec1 = pl.BlockSpec((1, ROW_BLK, H), lambda i: (1, i, 0))
    return pl.pallas_call(
        functools.partial(_tail_head_body, use_bn=use_bn),
        grid=(GRID_N,),
        in_specs=[p_spec0, p_spec1, blk_spec, vec_spec, vec_spec, vec_spec,
                  mat_spec, vec_spec, mat_spec],
        out_specs=blk_spec,
        out_shape=jax.ShapeDtypeStruct((N, H), jnp.float32),
    )(p, p, y, row(b1), row(gamma), row(beta), w2, row(b2), w1_next)


def _final_body(p0_ref, p1_ref, y_ref, b1_ref, w2_ref, b2_ref, batch_ref,
                mw1_ref, mb1_ref, mw2_ref, mb2_ref, mw3_ref, mb3_ref,
                o_ref, acc_ref):
    i = pl.program_id(0)
    t = p0_ref[0] + p1_ref[0] - y_ref[...] + b1_ref[...]
    t = _elu(t)
    t = jnp.dot(t, w2_ref[...], preferred_element_type=jnp.float32)
    z = _elu(t + b2_ref[...])
    b = batch_ref[0, 0, :]
    onehot_t = (lax.broadcasted_iota(jnp.int32, (G, ROW_BLK), 0)
                == b[None, :]).astype(jnp.float32)
    part = jnp.dot(onehot_t, z, preferred_element_type=jnp.float32)

    @pl.when(i == 0)
    def _():
        acc_ref[...] = jnp.zeros_like(acc_ref)

    acc_ref[...] += part

    @pl.when(i == GRID_N - 1)
    def _():
        pooled = acc_ref[...]
        r = _elu(jnp.dot(pooled, mw1_ref[...],
                         preferred_element_type=jnp.float32) + mb1_ref[...])
        r = _elu(jnp.dot(r, mw2_ref[...],
                         preferred_element_type=jnp.float32) + mb2_ref[...])
        o_ref[...] = jnp.dot(r, mw3_ref[...],
                             preferred_element_type=jnp.float32) + mb3_ref[...]


def _final(p, y, b1, w2, b2, batch3, mw1, mb1, mw2, mb2, mw3, mb3):
    vec = lambda v: v.reshape(1, -1)
    vec_spec = lambda n: pl.BlockSpec((1, n), lambda i: (0, 0))
    mat_spec = lambda m, n: pl.BlockSpec((m, n), lambda i: (0, 0))
    blk_spec = pl.BlockSpec((ROW_BLK, H), lambda i: (i, 0))
    return pl.pallas_call(
        _final_body,
        grid=(GRID_N,),
        in_specs=[
            pl.BlockSpec((1, ROW_BLK, H), lambda i: (0, i, 0)),
            pl.BlockSpec((1, ROW_BLK, H), lambda i: (1, i, 0)),
            blk_spec,
            vec_spec(H), mat_spec(H, H), vec_spec(H),
            pl.BlockSpec((1, 1, ROW_BLK), lambda i: (i, 0, 0)),
            mat_spec(H, H), vec_spec(H),
            mat_spec(H, H // 2), vec_spec(H // 2),
            mat_spec(H // 2, C), vec_spec(C),
        ],
        out_specs=pl.BlockSpec((G, C), lambda i: (0, 0)),
        out_shape=jax.ShapeDtypeStruct((G, C), jnp.float32),
        scratch_shapes=[pltpu.VMEM((G, H), jnp.float32)],
    )(p, p, y, vec(b1), w2, vec(b2), batch3, mw1, vec(mb1), mw2, vec(mb2),
      mw3, vec(mb3))


def kernel(x, edge_index, batch,
           pre0_w1, pre0_b1, pre0_gamma, pre0_beta, pre0_w2, pre0_b2,
           pre1_w1, pre1_b1, pre1_gamma, pre1_beta, pre1_w2, pre1_b2,
           post0_w1, post0_b1, post0_w2, post0_b2,
           mlp_w1, mlp_b1, mlp_w2, mlp_b2, mlp_w3, mlp_b3):
    pad_src = jnp.zeros((E_PAD,), jnp.int32)
    pad_dst = N + (jnp.arange(E_PAD, dtype=jnp.int32) % 128)
    srcs = jnp.concatenate([edge_index[0], pad_src]).reshape(NW, NCHUNKS, CHUNK)
    dsts = jnp.concatenate([edge_index[1], pad_dst]).reshape(NW, NCHUNKS, CHUNK)
    batch3 = batch.reshape(GRID_N, 1, ROW_BLK)

    y0 = _head_matmul(x, pre0_w1)
    p0 = _sc_aggregate(y0, srcs, dsts)
    y1 = _tail_head(p0, y0, pre0_b1, pre0_gamma, pre0_beta, pre0_w2, pre0_b2,
                    pre1_w1, use_bn=True)
    p1 = _sc_aggregate(y1, srcs, dsts)
    y2 = _tail_head(p1, y1, pre1_b1, pre1_gamma, pre1_beta, pre1_w2, pre1_b2,
                    post0_w1, use_bn=True)
    p2 = _sc_aggregate(y2, srcs, dsts)
    out = _final(p2, y2, post0_b1, post0_w2, post0_b2, batch3,
                 mlp_w1, mlp_b1, mlp_w2, mlp_b2, mlp_w3, mlp_b3)
    return (out, edge_index, batch)


# speedup vs baseline: 1.6513x; 1.0078x over previous
import functools

import jax
import jax.numpy as jnp
from jax import lax
from jax.experimental import pallas as pl
from jax.experimental.pallas import tpu as pltpu
from jax.experimental.pallas import tpu_sc as plsc

N = 10000
E = 320000
D = 128
H = 64
G = 32
C = 10

NC = 2
NS = 16
NW = NC * NS
CHUNK = 120
NCHUNKS = 84
EDGES_PER_W = NCHUNKS * CHUNK
E_PAD = NW * EDGES_PER_W - E
ROWS_A = 624
ROWS_REM = N - NS * ROWS_A

ROW_BLK = 2000
GRID_N = N // ROW_BLK

_BN_SCALE = 1.0 / (1.0 + 1e-5) ** 0.5


def _elu(t):
    return jnp.where(t > 0, t, jnp.exp(jnp.minimum(t, 0.0)) - 1.0)


def _agg_body(y_hbm, srcs_hbm, dsts_hbm, out_hbm, src_v, dst_v, buf0, buf1,
              acc_sh, gsem, ssem):
    c = lax.axis_index("c")
    s = lax.axis_index("s")
    wid = c * NS + s
    pltpu.sync_copy(y_hbm.at[pl.ds(s * ROWS_A, ROWS_A)],
                    acc_sh.at[pl.ds(s * ROWS_A, ROWS_A)])

    @pl.when(s == NS - 1)
    def _():
        pltpu.sync_copy(y_hbm.at[pl.ds(NS * ROWS_A, ROWS_REM)],
                        acc_sh.at[pl.ds(NS * ROWS_A, ROWS_REM)])
    pltpu.sync_copy(srcs_hbm.at[wid], src_v)
    pltpu.sync_copy(dsts_hbm.at[wid], dst_v)
    plsc.subcore_barrier()

    def g_fire(j, buf):
        pltpu.async_copy(y_hbm.at[src_v.at[j]], buf, gsem)

    def g_wait(j, buf):
        pltpu.make_async_copy(y_hbm.at[src_v.at[j]], buf, gsem).wait()

    def s_sync(j, buf):
        pltpu.sync_copy(buf, acc_sh.at[dst_v.at[j]], add=True)

    g_fire(0, buf0)

    def body(t, carry):
        j0 = 2 * t
        j1 = j0 + 1
        g_fire(j1, buf1)
        g_wait(j0, buf0)
        s_sync(j0, buf0)

        @pl.when(t < NCHUNKS // 2 - 1)
        def _():
            g_fire(j0 + 2, buf0)
        g_wait(j1, buf1)
        s_sync(j1, buf1)
        return carry

    lax.fori_loop(0, NCHUNKS // 2, body, 0)
    plsc.subcore_barrier()
    pltpu.sync_copy(acc_sh.at[pl.ds(s * ROWS_A, ROWS_A)],
                    out_hbm.at[c, pl.ds(s * ROWS_A, ROWS_A)])

    @pl.when(s == NS - 1)
    def _():
        pltpu.sync_copy(acc_sh.at[pl.ds(NS * ROWS_A, ROWS_REM)],
                        out_hbm.at[c, pl.ds(NS * ROWS_A, ROWS_REM)])


def _make_agg():
    mesh = plsc.VectorSubcoreMesh(core_axis_name="c", subcore_axis_name="s",
                                  num_cores=NC, num_subcores=NS)
    return pl.kernel(
        _agg_body,
        out_type=jax.ShapeDtypeStruct((NC, N, H), jnp.float32),
        mesh=mesh,
        scratch_types=[
            pltpu.VMEM((NCHUNKS, CHUNK), jnp.int32),
            pltpu.VMEM((NCHUNKS, CHUNK), jnp.int32),
            pltpu.VMEM((CHUNK, H), jnp.float32),
            pltpu.VMEM((CHUNK, H), jnp.float32),
            pltpu.VMEM_SHARED((N + 128, H), jnp.float32),
            pltpu.SemaphoreType.DMA,
            pltpu.SemaphoreType.DMA,
        ],
        compiler_params=pltpu.CompilerParams(use_tc_tiling_on_sc=False),
    )


_sc_aggregate = _make_agg()


def _head_body(x_ref, w_ref, o_ref):
    o_ref[...] = jnp.dot(x_ref[...], w_ref[...],
                         preferred_element_type=jnp.float32)


def _head_matmul(x, w):
    d_in = x.shape[1]
    return pl.pallas_call(
        _head_body,
        grid=(GRID_N,),
        in_specs=[
            pl.BlockSpec((ROW_BLK, d_in), lambda i: (i, 0)),
            pl.BlockSpec((d_in, H), lambda i: (0, 0)),
        ],
        out_specs=pl.BlockSpec((ROW_BLK, H), lambda i: (i, 0)),
        out_shape=jax.ShapeDtypeStruct((N, H), jnp.float32),
    )(x, w)


def _tail_head_body(p0_ref, p1_ref, y_ref, b1_ref, g_ref, bt_ref, w2_ref,
                    b2_ref, w1n_ref, o_ref, *, use_bn):
    t = p0_ref[0] + p1_ref[0] - y_ref[...] + b1_ref[...]
    if use_bn:
        t = t * (_BN_SCALE * g_ref[...]) + bt_ref[...]
    t = _elu(t)
    t = jnp.dot(t, w2_ref[...], preferred_element_type=jnp.float32)
    z = _elu(t + b2_ref[...])
    o_ref[...] = jnp.dot(z, w1n_ref[...], preferred_element_type=jnp.float32)


def _tail_head(p, y, b1, gamma, beta, w2, b2, w1_next, use_bn):
    row = lambda v: v.reshape(1, H)
    vec_spec = pl.BlockSpec((1, H), lambda i: (0, 0))
    mat_spec = pl.BlockSpec((H, H), lambda i: (0, 0))
    blk_spec = pl.BlockSpec((ROW_BLK, H), lambda i: (i, 0))
    p_spec0 = pl.BlockSpec((1, ROW_BLK, H), lambda i: (0, i, 0))
    p_spec1 = pl.BlockSpec((1, ROW_BLK, H), lambda i: (1, i, 0))
    return pl.pallas_call(
        functools.partial(_tail_head_body, use_bn=use_bn),
        grid=(GRID_N,),
        in_specs=[p_spec0, p_spec1, blk_spec, vec_spec, vec_spec, vec_spec,
                  mat_spec, vec_spec, mat_spec],
        out_specs=blk_spec,
        out_shape=jax.ShapeDtypeStruct((N, H), jnp.float32),
    )(p, p, y, row(b1), row(gamma), row(beta), w2, row(b2), w1_next)


def _final_body(p0_ref, p1_ref, y_ref, b1_ref, w2_ref, b2_ref, batch_ref,
                mw1_ref, mb1_ref, mw2_ref, mb2_ref, mw3_ref, mb3_ref,
                o_ref, acc_ref):
    i = pl.program_id(0)
    t = p0_ref[0] + p1_ref[0] - y_ref[...] + b1_ref[...]
    t = _elu(t)
    t = jnp.dot(t, w2_ref[...], preferred_element_type=jnp.float32)
    z = _elu(t + b2_ref[...])
    b = batch_ref[0, 0, :]
    onehot_t = (lax.broadcasted_iota(jnp.int32, (G, ROW_BLK), 0)
                == b[None, :]).astype(jnp.float32)
    part = jnp.dot(onehot_t, z, preferred_element_type=jnp.float32)

    @pl.when(i == 0)
    def _():
        acc_ref[...] = jnp.zeros_like(acc_ref)

    acc_ref[...] += part

    @pl.when(i == GRID_N - 1)
    def _():
        pooled = acc_ref[...]
        r = _elu(jnp.dot(pooled, mw1_ref[...],
                         preferred_element_type=jnp.float32) + mb1_ref[...])
        r = _elu(jnp.dot(r, mw2_ref[...],
                         preferred_element_type=jnp.float32) + mb2_ref[...])
        o_ref[...] = jnp.dot(r, mw3_ref[...],
                             preferred_element_type=jnp.float32) + mb3_ref[...]


def _final(p, y, b1, w2, b2, batch3, mw1, mb1, mw2, mb2, mw3, mb3):
    vec = lambda v: v.reshape(1, -1)
    vec_spec = lambda n: pl.BlockSpec((1, n), lambda i: (0, 0))
    mat_spec = lambda m, n: pl.BlockSpec((m, n), lambda i: (0, 0))
    blk_spec = pl.BlockSpec((ROW_BLK, H), lambda i: (i, 0))
    return pl.pallas_call(
        _final_body,
        grid=(GRID_N,),
        in_specs=[
            pl.BlockSpec((1, ROW_BLK, H), lambda i: (0, i, 0)),
            pl.BlockSpec((1, ROW_BLK, H), lambda i: (1, i, 0)),
            blk_spec,
            vec_spec(H), mat_spec(H, H), vec_spec(H),
            pl.BlockSpec((1, 1, ROW_BLK), lambda i: (i, 0, 0)),
            mat_spec(H, H), vec_spec(H),
            mat_spec(H, H // 2), vec_spec(H // 2),
            mat_spec(H // 2, C), vec_spec(C),
        ],
        out_specs=pl.BlockSpec((G, C), lambda i: (0, 0)),
        out_shape=jax.ShapeDtypeStruct((G, C), jnp.float32),
        scratch_shapes=[pltpu.VMEM((G, H), jnp.float32)],
    )(p, p, y, vec(b1), w2, vec(b2), batch3, mw1, vec(mb1), mw2, vec(mb2),
      mw3, vec(mb3))


def kernel(x, edge_index, batch,
           pre0_w1, pre0_b1, pre0_gamma, pre0_beta, pre0_w2, pre0_b2,
           pre1_w1, pre1_b1, pre1_gamma, pre1_beta, pre1_w2, pre1_b2,
           post0_w1, post0_b1, post0_w2, post0_b2,
           mlp_w1, mlp_b1, mlp_w2, mlp_b2, mlp_w3, mlp_b3):
    pad_src = jnp.zeros((E_PAD,), jnp.int32)
    pad_dst = N + (jnp.arange(E_PAD, dtype=jnp.int32) % 128)
    srcs = jnp.concatenate([edge_index[0], pad_src]).reshape(NW, NCHUNKS, CHUNK)
    dsts = jnp.concatenate([edge_index[1], pad_dst]).reshape(NW, NCHUNKS, CHUNK)
    batch3 = batch.reshape(GRID_N, 1, ROW_BLK)

    y0 = _head_matmul(x, pre0_w1)
    p0 = _sc_aggregate(y0, srcs, dsts)
    y1 = _tail_head(p0, y0, pre0_b1, pre0_gamma, pre0_beta, pre0_w2, pre0_b2,
                    pre1_w1, use_bn=True)
    p1 = _sc_aggregate(y1, srcs, dsts)
    y2 = _tail_head(p1, y1, pre1_b1, pre1_gamma, pre1_beta, pre1_w2, pre1_b2,
                    post0_w1, use_bn=True)
    p2 = _sc_aggregate(y2, srcs, dsts)
    out = _final(p2, y2, post0_b1, post0_w2, post0_b2, batch3,
                 mlp_w1, mlp_b1, mlp_w2, mlp_b2, mlp_w3, mlp_b3)
    return (out, edge_index, batch)


# speedup vs baseline: 1.7201x; 1.0417x over previous
import functools

import jax
import jax.numpy as jnp
from jax import lax
from jax.experimental import pallas as pl
from jax.experimental.pallas import tpu as pltpu
from jax.experimental.pallas import tpu_sc as plsc

N = 10000
E = 320000
D = 128
H = 64
G = 32
C = 10

NC = 2
NS = 16
NW = NC * NS
CHUNK = 120
FAST_C = 0
CNT_F = 106
CNT_S = 62
E_FAST = NS * CNT_F * CHUNK
E_SLOW = NS * CNT_S * CHUNK
E_PAD = E_FAST + E_SLOW - E
ROWS_A = 624
ROWS_REM = N - NS * ROWS_A

ROW_BLK = 2000
GRID_N = N // ROW_BLK

_BN_SCALE = 1.0 / (1.0 + 1e-5) ** 0.5


def _elu(t):
    return jnp.where(t > 0, t, jnp.exp(jnp.minimum(t, 0.0)) - 1.0)


def _agg_body(y_hbm, srcs_hbm, dsts_hbm, out_hbm, src_v, dst_v, buf0, buf1,
              acc_sh, gsem, ssem):
    c = lax.axis_index("c")
    s = lax.axis_index("s")
    wid = c * NS + s
    pltpu.sync_copy(y_hbm.at[pl.ds(s * ROWS_A, ROWS_A)],
                    acc_sh.at[pl.ds(s * ROWS_A, ROWS_A)])

    @pl.when(s == NS - 1)
    def _():
        pltpu.sync_copy(y_hbm.at[pl.ds(NS * ROWS_A, ROWS_REM)],
                        acc_sh.at[pl.ds(NS * ROWS_A, ROWS_REM)])
    pltpu.sync_copy(srcs_hbm.at[wid], src_v)
    pltpu.sync_copy(dsts_hbm.at[wid], dst_v)
    cnt = jnp.where(c == FAST_C, CNT_F, CNT_S)
    plsc.subcore_barrier()

    def g_fire(j, buf):
        pltpu.async_copy(y_hbm.at[src_v.at[j]], buf, gsem)

    def g_wait(j, buf):
        pltpu.make_async_copy(y_hbm.at[src_v.at[j]], buf, gsem).wait()

    def s_sync(j, buf):
        pltpu.sync_copy(buf, acc_sh.at[dst_v.at[j]], add=True)

    g_fire(0, buf0)

    def body(t, carry):
        j0 = 2 * t
        j1 = j0 + 1
        g_fire(j1, buf1)
        g_wait(j0, buf0)
        s_sync(j0, buf0)

        @pl.when(t < cnt // 2 - 1)
        def _():
            g_fire(j0 + 2, buf0)
        g_wait(j1, buf1)
        s_sync(j1, buf1)
        return carry

    lax.fori_loop(0, cnt // 2, body, 0)
    plsc.subcore_barrier()
    pltpu.sync_copy(acc_sh.at[pl.ds(s * ROWS_A, ROWS_A)],
                    out_hbm.at[c, pl.ds(s * ROWS_A, ROWS_A)])

    @pl.when(s == NS - 1)
    def _():
        pltpu.sync_copy(acc_sh.at[pl.ds(NS * ROWS_A, ROWS_REM)],
                        out_hbm.at[c, pl.ds(NS * ROWS_A, ROWS_REM)])


def _make_agg():
    mesh = plsc.VectorSubcoreMesh(core_axis_name="c", subcore_axis_name="s",
                                  num_cores=NC, num_subcores=NS)
    return pl.kernel(
        _agg_body,
        out_type=jax.ShapeDtypeStruct((NC, N, H), jnp.float32),
        mesh=mesh,
        scratch_types=[
            pltpu.VMEM((CNT_F, CHUNK), jnp.int32),
            pltpu.VMEM((CNT_F, CHUNK), jnp.int32),
            pltpu.VMEM((CHUNK, H), jnp.float32),
            pltpu.VMEM((CHUNK, H), jnp.float32),
            pltpu.VMEM_SHARED((N + 128, H), jnp.float32),
            pltpu.SemaphoreType.DMA,
            pltpu.SemaphoreType.DMA,
        ],
        compiler_params=pltpu.CompilerParams(use_tc_tiling_on_sc=False),
    )


_sc_aggregate = _make_agg()


def _head_body(x_ref, w_ref, o_ref):
    o_ref[...] = jnp.dot(x_ref[...], w_ref[...],
                         preferred_element_type=jnp.float32)


def _head_matmul(x, w):
    d_in = x.shape[1]
    return pl.pallas_call(
        _head_body,
        grid=(GRID_N,),
        in_specs=[
            pl.BlockSpec((ROW_BLK, d_in), lambda i: (i, 0)),
            pl.BlockSpec((d_in, H), lambda i: (0, 0)),
        ],
        out_specs=pl.BlockSpec((ROW_BLK, H), lambda i: (i, 0)),
        out_shape=jax.ShapeDtypeStruct((N, H), jnp.float32),
    )(x, w)


def _tail_head_body(p0_ref, p1_ref, y_ref, b1_ref, g_ref, bt_ref, w2_ref,
                    b2_ref, w1n_ref, o_ref, *, use_bn):
    t = p0_ref[0] + p1_ref[0] - y_ref[...] + b1_ref[...]
    if use_bn:
        t = t * (_BN_SCALE * g_ref[...]) + bt_ref[...]
    t = _elu(t)
    t = jnp.dot(t, w2_ref[...], preferred_element_type=jnp.float32)
    z = _elu(t + b2_ref[...])
    o_ref[...] = jnp.dot(z, w1n_ref[...], preferred_element_type=jnp.float32)


def _tail_head(p, y, b1, gamma, beta, w2, b2, w1_next, use_bn):
    row = lambda v: v.reshape(1, H)
    vec_spec = pl.BlockSpec((1, H), lambda i: (0, 0))
    mat_spec = pl.BlockSpec((H, H), lambda i: (0, 0))
    blk_spec = pl.BlockSpec((ROW_BLK, H), lambda i: (i, 0))
    p_spec0 = pl.BlockSpec((1, ROW_BLK, H), lambda i: (0, i, 0))
    p_spec1 = pl.BlockSpec((1, ROW_BLK, H), lambda i: (1, i, 0))
    return pl.pallas_call(
        functools.partial(_tail_head_body, use_bn=use_bn),
        grid=(GRID_N,),
        in_specs=[p_spec0, p_spec1, blk_spec, vec_spec, vec_spec, vec_spec,
                  mat_spec, vec_spec, mat_spec],
        out_specs=blk_spec,
        out_shape=jax.ShapeDtypeStruct((N, H), jnp.float32),
    )(p, p, y, row(b1), row(gamma), row(beta), w2, row(b2), w1_next)


def _final_body(p0_ref, p1_ref, y_ref, b1_ref, w2_ref, b2_ref, batch_ref,
                mw1_ref, mb1_ref, mw2_ref, mb2_ref, mw3_ref, mb3_ref,
                o_ref, acc_ref):
    i = pl.program_id(0)
    t = p0_ref[0] + p1_ref[0] - y_ref[...] + b1_ref[...]
    t = _elu(t)
    t = jnp.dot(t, w2_ref[...], preferred_element_type=jnp.float32)
    z = _elu(t + b2_ref[...])
    b = batch_ref[0, 0, :]
    onehot_t = (lax.broadcasted_iota(jnp.int32, (G, ROW_BLK), 0)
                == b[None, :]).astype(jnp.float32)
    part = jnp.dot(onehot_t, z, preferred_element_type=jnp.float32)

    @pl.when(i == 0)
    def _():
        acc_ref[...] = jnp.zeros_like(acc_ref)

    acc_ref[...] += part

    @pl.when(i == GRID_N - 1)
    def _():
        pooled = acc_ref[...]
        r = _elu(jnp.dot(pooled, mw1_ref[...],
                         preferred_element_type=jnp.float32) + mb1_ref[...])
        r = _elu(jnp.dot(r, mw2_ref[...],
                         preferred_element_type=jnp.float32) + mb2_ref[...])
        o_ref[...] = jnp.dot(r, mw3_ref[...],
                             preferred_element_type=jnp.float32) + mb3_ref[...]


def _final(p, y, b1, w2, b2, batch3, mw1, mb1, mw2, mb2, mw3, mb3):
    vec = lambda v: v.reshape(1, -1)
    vec_spec = lambda n: pl.BlockSpec((1, n), lambda i: (0, 0))
    mat_spec = lambda m, n: pl.BlockSpec((m, n), lambda i: (0, 0))
    blk_spec = pl.BlockSpec((ROW_BLK, H), lambda i: (i, 0))
    return pl.pallas_call(
        _final_body,
        grid=(GRID_N,),
        in_specs=[
            pl.BlockSpec((1, ROW_BLK, H), lambda i: (0, i, 0)),
            pl.BlockSpec((1, ROW_BLK, H), lambda i: (1, i, 0)),
            blk_spec,
            vec_spec(H), mat_spec(H, H), vec_spec(H),
            pl.BlockSpec((1, 1, ROW_BLK), lambda i: (i, 0, 0)),
            mat_spec(H, H), vec_spec(H),
            mat_spec(H, H // 2), vec_spec(H // 2),
            mat_spec(H // 2, C), vec_spec(C),
        ],
        out_specs=pl.BlockSpec((G, C), lambda i: (0, 0)),
        out_shape=jax.ShapeDtypeStruct((G, C), jnp.float32),
        scratch_shapes=[pltpu.VMEM((G, H), jnp.float32)],
    )(p, p, y, vec(b1), w2, vec(b2), batch3, mw1, vec(mb1), mw2, vec(mb2),
      mw3, vec(mb3))


def kernel(x, edge_index, batch,
           pre0_w1, pre0_b1, pre0_gamma, pre0_beta, pre0_w2, pre0_b2,
           pre1_w1, pre1_b1, pre1_gamma, pre1_beta, pre1_w2, pre1_b2,
           post0_w1, post0_b1, post0_w2, post0_b2,
           mlp_w1, mlp_b1, mlp_w2, mlp_b2, mlp_w3, mlp_b3):
    def slabs(ep):
        fast = ep[:E_FAST].reshape(NS, CNT_F * CHUNK)
        slow = jnp.concatenate(
            [ep[E_FAST:].reshape(NS, CNT_S * CHUNK),
             jnp.zeros((NS, (CNT_F - CNT_S) * CHUNK), jnp.int32)], axis=1)
        both = (fast, slow) if FAST_C == 0 else (slow, fast)
        return jnp.concatenate(both, axis=0).reshape(NW, CNT_F, CHUNK)

    pad_dst = N + (jnp.arange(E_PAD, dtype=jnp.int32) % 128)
    srcs = slabs(jnp.concatenate([edge_index[0], jnp.zeros((E_PAD,), jnp.int32)]))
    dsts = slabs(jnp.concatenate([edge_index[1], pad_dst]))
    batch3 = batch.reshape(GRID_N, 1, ROW_BLK)

    y0 = _head_matmul(x, pre0_w1)
    p0 = _sc_aggregate(y0, srcs, dsts)
    y1 = _tail_head(p0, y0, pre0_b1, pre0_gamma, pre0_beta, pre0_w2, pre0_b2,
                    pre1_w1, use_bn=True)
    p1 = _sc_aggregate(y1, srcs, dsts)
    y2 = _tail_head(p1, y1, pre1_b1, pre1_gamma, pre1_beta, pre1_w2, pre1_b2,
                    post0_w1, use_bn=True)
    p2 = _sc_aggregate(y2, srcs, dsts)
    out = _final(p2, y2, post0_b1, post0_w2, post0_b2, batch3,
                 mlp_w1, mlp_b1, mlp_w2, mlp_b2, mlp_w3, mlp_b3)
    return (out, edge_index, batch)


# speedup vs baseline: 1.8907x; 1.0992x over previous
import functools

import jax
import jax.numpy as jnp
from jax import lax
from jax.experimental import pallas as pl
from jax.experimental.pallas import tpu as pltpu
from jax.experimental.pallas import tpu_sc as plsc

N = 10000
E = 320000
D = 128
H = 64
G = 32
C = 10

NC = 2
NS = 16
NW = NC * NS
CHUNK = 120
FAST_C = 0
CNT_F = 106
CNT_S = 62
E_FAST = NS * CNT_F * CHUNK
E_SLOW = NS * CNT_S * CHUNK
E_PAD = E_FAST + E_SLOW - E
ROWS_A = 624
ROWS_REM = N - NS * ROWS_A

ROW_BLK = 2000
GRID_N = N // ROW_BLK

_BN_SCALE = 1.0 / (1.0 + 1e-5) ** 0.5


def _elu(t):
    return jnp.where(t > 0, t, jnp.exp(jnp.minimum(t, 0.0)) - 1.0)


def _agg_body(y_hbm, srcs_hbm, dsts_hbm, out_hbm, src_v, dst_v, buf0, buf1,
              acc_sh, tbl_sh, gsem, ssem):
    c = lax.axis_index("c")
    s = lax.axis_index("s")
    wid = c * NS + s
    pltpu.sync_copy(y_hbm.at[pl.ds(s * ROWS_A, ROWS_A)],
                    acc_sh.at[pl.ds(s * ROWS_A, ROWS_A)])
    pltpu.sync_copy(y_hbm.at[pl.ds(s * ROWS_A, ROWS_A)],
                    tbl_sh.at[pl.ds(s * ROWS_A, ROWS_A)])

    @pl.when(s == NS - 1)
    def _():
        pltpu.sync_copy(y_hbm.at[pl.ds(NS * ROWS_A, ROWS_REM)],
                        acc_sh.at[pl.ds(NS * ROWS_A, ROWS_REM)])
        pltpu.sync_copy(y_hbm.at[pl.ds(NS * ROWS_A, ROWS_REM)],
                        tbl_sh.at[pl.ds(NS * ROWS_A, ROWS_REM)])
    pltpu.sync_copy(srcs_hbm.at[wid], src_v)
    pltpu.sync_copy(dsts_hbm.at[wid], dst_v)
    cnt = jnp.where(c == FAST_C, CNT_F, CNT_S)
    plsc.subcore_barrier()

    def g_fire(j, buf):
        pltpu.async_copy(tbl_sh.at[src_v.at[j]], buf, gsem)

    def g_wait(j, buf):
        pltpu.make_async_copy(tbl_sh.at[src_v.at[j]], buf, gsem).wait()

    def s_sync(j, buf):
        pltpu.sync_copy(buf, acc_sh.at[dst_v.at[j]], add=True)

    g_fire(0, buf0)

    def body(t, carry):
        j0 = 2 * t
        j1 = j0 + 1
        g_fire(j1, buf1)
        g_wait(j0, buf0)
        s_sync(j0, buf0)

        @pl.when(t < cnt // 2 - 1)
        def _():
            g_fire(j0 + 2, buf0)
        g_wait(j1, buf1)
        s_sync(j1, buf1)
        return carry

    lax.fori_loop(0, cnt // 2, body, 0)
    plsc.subcore_barrier()
    pltpu.sync_copy(acc_sh.at[pl.ds(s * ROWS_A, ROWS_A)],
                    out_hbm.at[c, pl.ds(s * ROWS_A, ROWS_A)])

    @pl.when(s == NS - 1)
    def _():
        pltpu.sync_copy(acc_sh.at[pl.ds(NS * ROWS_A, ROWS_REM)],
                        out_hbm.at[c, pl.ds(NS * ROWS_A, ROWS_REM)])


def _make_agg():
    mesh = plsc.VectorSubcoreMesh(core_axis_name="c", subcore_axis_name="s",
                                  num_cores=NC, num_subcores=NS)
    return pl.kernel(
        _agg_body,
        out_type=jax.ShapeDtypeStruct((NC, N, H), jnp.float32),
        mesh=mesh,
        scratch_types=[
            pltpu.VMEM((CNT_F, CHUNK), jnp.int32),
            pltpu.VMEM((CNT_F, CHUNK), jnp.int32),
            pltpu.VMEM((CHUNK, H), jnp.float32),
            pltpu.VMEM((CHUNK, H), jnp.float32),
            pltpu.VMEM_SHARED((N + 128, H), jnp.float32),
            pltpu.VMEM_SHARED((N, H), jnp.float32),
            pltpu.SemaphoreType.DMA,
            pltpu.SemaphoreType.DMA,
        ],
        compiler_params=pltpu.CompilerParams(use_tc_tiling_on_sc=False),
    )


_sc_aggregate = _make_agg()


def _head_body(x_ref, w_ref, o_ref):
    o_ref[...] = jnp.dot(x_ref[...], w_ref[...],
                         preferred_element_type=jnp.float32)


def _head_matmul(x, w):
    d_in = x.shape[1]
    return pl.pallas_call(
        _head_body,
        grid=(GRID_N,),
        in_specs=[
            pl.BlockSpec((ROW_BLK, d_in), lambda i: (i, 0)),
            pl.BlockSpec((d_in, H), lambda i: (0, 0)),
        ],
        out_specs=pl.BlockSpec((ROW_BLK, H), lambda i: (i, 0)),
        out_shape=jax.ShapeDtypeStruct((N, H), jnp.float32),
    )(x, w)


def _tail_head_body(p0_ref, p1_ref, y_ref, b1_ref, g_ref, bt_ref, w2_ref,
                    b2_ref, w1n_ref, o_ref, *, use_bn):
    t = p0_ref[0] + p1_ref[0] - y_ref[...] + b1_ref[...]
    if use_bn:
        t = t * (_BN_SCALE * g_ref[...]) + bt_ref[...]
    t = _elu(t)
    t = jnp.dot(t, w2_ref[...], preferred_element_type=jnp.float32)
    z = _elu(t + b2_ref[...])
    o_ref[...] = jnp.dot(z, w1n_ref[...], preferred_element_type=jnp.float32)


def _tail_head(p, y, b1, gamma, beta, w2, b2, w1_next, use_bn):
    row = lambda v: v.reshape(1, H)
    vec_spec = pl.BlockSpec((1, H), lambda i: (0, 0))
    mat_spec = pl.BlockSpec((H, H), lambda i: (0, 0))
    blk_spec = pl.BlockSpec((ROW_BLK, H), lambda i: (i, 0))
    p_spec0 = pl.BlockSpec((1, ROW_BLK, H), lambda i: (0, i, 0))
    p_spec1 = pl.BlockSpec((1, ROW_BLK, H), lambda i: (1, i, 0))
    return pl.pallas_call(
        functools.partial(_tail_head_body, use_bn=use_bn),
        grid=(GRID_N,),
        in_specs=[p_spec0, p_spec1, blk_spec, vec_spec, vec_spec, vec_spec,
                  mat_spec, vec_spec, mat_spec],
        out_specs=blk_spec,
        out_shape=jax.ShapeDtypeStruct((N, H), jnp.float32),
    )(p, p, y, row(b1), row(gamma), row(beta), w2, row(b2), w1_next)


def _final_body(p0_ref, p1_ref, y_ref, b1_ref, w2_ref, b2_ref, batch_ref,
                mw1_ref, mb1_ref, mw2_ref, mb2_ref, mw3_ref, mb3_ref,
                o_ref, acc_ref):
    i = pl.program_id(0)
    t = p0_ref[0] + p1_ref[0] - y_ref[...] + b1_ref[...]
    t = _elu(t)
    t = jnp.dot(t, w2_ref[...], preferred_element_type=jnp.float32)
    z = _elu(t + b2_ref[...])
    b = batch_ref[0, 0, :]
    onehot_t = (lax.broadcasted_iota(jnp.int32, (G, ROW_BLK), 0)
                == b[None, :]).astype(jnp.float32)
    part = jnp.dot(onehot_t, z, preferred_element_type=jnp.float32)

    @pl.when(i == 0)
    def _():
        acc_ref[...] = jnp.zeros_like(acc_ref)

    acc_ref[...] += part

    @pl.when(i == GRID_N - 1)
    def _():
        pooled = acc_ref[...]
        r = _elu(jnp.dot(pooled, mw1_ref[...],
                         preferred_element_type=jnp.float32) + mb1_ref[...])
        r = _elu(jnp.dot(r, mw2_ref[...],
                         preferred_element_type=jnp.float32) + mb2_ref[...])
        o_ref[...] = jnp.dot(r, mw3_ref[...],
                             preferred_element_type=jnp.float32) + mb3_ref[...]


def _final(p, y, b1, w2, b2, batch3, mw1, mb1, mw2, mb2, mw3, mb3):
    vec = lambda v: v.reshape(1, -1)
    vec_spec = lambda n: pl.BlockSpec((1, n), lambda i: (0, 0))
    mat_spec = lambda m, n: pl.BlockSpec((m, n), lambda i: (0, 0))
    blk_spec = pl.BlockSpec((ROW_BLK, H), lambda i: (i, 0))
    return pl.pallas_call(
        _final_body,
        grid=(GRID_N,),
        in_specs=[
            pl.BlockSpec((1, ROW_BLK, H), lambda i: (0, i, 0)),
            pl.BlockSpec((1, ROW_BLK, H), lambda i: (1, i, 0)),
            blk_spec,
            vec_spec(H), mat_spec(H, H), vec_spec(H),
            pl.BlockSpec((1, 1, ROW_BLK), lambda i: (i, 0, 0)),
            mat_spec(H, H), vec_spec(H),
            mat_spec(H, H // 2), vec_spec(H // 2),
            mat_spec(H // 2, C), vec_spec(C),
        ],
        out_specs=pl.BlockSpec((G, C), lambda i: (0, 0)),
        out_shape=jax.ShapeDtypeStruct((G, C), jnp.float32),
        scratch_shapes=[pltpu.VMEM((G, H), jnp.float32)],
    )(p, p, y, vec(b1), w2, vec(b2), batch3, mw1, vec(mb1), mw2, vec(mb2),
      mw3, vec(mb3))


def kernel(x, edge_index, batch,
           pre0_w1, pre0_b1, pre0_gamma, pre0_beta, pre0_w2, pre0_b2,
           pre1_w1, pre1_b1, pre1_gamma, pre1_beta, pre1_w2, pre1_b2,
           post0_w1, post0_b1, post0_w2, post0_b2,
           mlp_w1, mlp_b1, mlp_w2, mlp_b2, mlp_w3, mlp_b3):
    def slabs(ep):
        fast = ep[:E_FAST].reshape(NS, CNT_F * CHUNK)
        slow = jnp.concatenate(
            [ep[E_FAST:].reshape(NS, CNT_S * CHUNK),
             jnp.zeros((NS, (CNT_F - CNT_S) * CHUNK), jnp.int32)], axis=1)
        both = (fast, slow) if FAST_C == 0 else (slow, fast)
        return jnp.concatenate(both, axis=0).reshape(NW, CNT_F, CHUNK)

    pad_dst = N + (jnp.arange(E_PAD, dtype=jnp.int32) % 128)
    srcs = slabs(jnp.concatenate([edge_index[0], jnp.zeros((E_PAD,), jnp.int32)]))
    dsts = slabs(jnp.concatenate([edge_index[1], pad_dst]))
    batch3 = batch.reshape(GRID_N, 1, ROW_BLK)

    y0 = _head_matmul(x, pre0_w1)
    p0 = _sc_aggregate(y0, srcs, dsts)
    y1 = _tail_head(p0, y0, pre0_b1, pre0_gamma, pre0_beta, pre0_w2, pre0_b2,
                    pre1_w1, use_bn=True)
    p1 = _sc_aggregate(y1, srcs, dsts)
    y2 = _tail_head(p1, y1, pre1_b1, pre1_gamma, pre1_beta, pre1_w2, pre1_b2,
                    post0_w1, use_bn=True)
    p2 = _sc_aggregate(y2, srcs, dsts)
    out = _final(p2, y2, post0_b1, post0_w2, post0_b2, batch3,
                 mlp_w1, mlp_b1, mlp_w2, mlp_b2, mlp_w3, mlp_b3)
    return (out, edge_index, batch)


# speedup vs baseline: 2.2358x; 1.1825x over previous
import functools

import jax
import jax.numpy as jnp
from jax import lax
from jax.experimental import pallas as pl
from jax.experimental.pallas import tpu as pltpu
from jax.experimental.pallas import tpu_sc as plsc

N = 10000
E = 320000
D = 128
H = 64
G = 32
C = 10

NC = 2
NS = 16
NW = NC * NS
CHUNK = 120
FAST_C = 0
CNT_F = 88
CNT_S = 80
CH_BASE_S = NS * CNT_F
TOTCH = NS * (CNT_F + CNT_S)
TOTCH_PAD = CH_BASE_S + (NS - 1) * CNT_S + CNT_F
E_PAD = TOTCH_PAD * CHUNK - E
ROWS_A = 624
ROWS_REM = N - NS * ROWS_A

ROW_BLK = 2000
GRID_N = N // ROW_BLK

_BN_SCALE = 1.0 / (1.0 + 1e-5) ** 0.5


def _elu(t):
    return jnp.where(t > 0, t, jnp.exp(jnp.minimum(t, 0.0)) - 1.0)


def _agg_body(y_hbm, srcs_hbm, dsts_hbm, out_hbm, src_v, dst_v, buf0, buf1,
              acc_sh, tbl_sh, gsem, ssem):
    c = lax.axis_index("c")
    s = lax.axis_index("s")
    wid = c * NS + s
    pltpu.sync_copy(y_hbm.at[pl.ds(s * ROWS_A, ROWS_A)],
                    acc_sh.at[pl.ds(s * ROWS_A, ROWS_A)])
    pltpu.sync_copy(y_hbm.at[pl.ds(s * ROWS_A, ROWS_A)],
                    tbl_sh.at[pl.ds(s * ROWS_A, ROWS_A)])

    @pl.when(s == NS - 1)
    def _():
        pltpu.sync_copy(y_hbm.at[pl.ds(NS * ROWS_A, ROWS_REM)],
                        acc_sh.at[pl.ds(NS * ROWS_A, ROWS_REM)])
        pltpu.sync_copy(y_hbm.at[pl.ds(NS * ROWS_A, ROWS_REM)],
                        tbl_sh.at[pl.ds(NS * ROWS_A, ROWS_REM)])
    chunk_off = jnp.where(c == FAST_C, s * CNT_F, CH_BASE_S + s * CNT_S)
    pltpu.sync_copy(srcs_hbm.at[pl.ds(chunk_off, CNT_F)], src_v)
    pltpu.sync_copy(dsts_hbm.at[pl.ds(chunk_off, CNT_F)], dst_v)
    cnt = jnp.where(c == FAST_C, CNT_F, CNT_S)
    plsc.subcore_barrier()

    def g_fire(j, buf):
        pltpu.async_copy(tbl_sh.at[src_v.at[j]], buf, gsem)

    def g_wait(j, buf):
        pltpu.make_async_copy(tbl_sh.at[src_v.at[j]], buf, gsem).wait()

    def s_sync(j, buf):
        pltpu.sync_copy(buf, acc_sh.at[dst_v.at[j]], add=True)

    g_fire(0, buf0)

    def body(t, carry):
        j0 = 2 * t
        j1 = j0 + 1
        g_fire(j1, buf1)
        g_wait(j0, buf0)
        s_sync(j0, buf0)

        @pl.when(t < cnt // 2 - 1)
        def _():
            g_fire(j0 + 2, buf0)
        g_wait(j1, buf1)
        s_sync(j1, buf1)
        return carry

    lax.fori_loop(0, cnt // 2, body, 0)
    plsc.subcore_barrier()
    pltpu.sync_copy(acc_sh.at[pl.ds(s * ROWS_A, ROWS_A)],
                    out_hbm.at[c, pl.ds(s * ROWS_A, ROWS_A)])

    @pl.when(s == NS - 1)
    def _():
        pltpu.sync_copy(acc_sh.at[pl.ds(NS * ROWS_A, ROWS_REM)],
                        out_hbm.at[c, pl.ds(NS * ROWS_A, ROWS_REM)])


def _make_agg():
    mesh = plsc.VectorSubcoreMesh(core_axis_name="c", subcore_axis_name="s",
                                  num_cores=NC, num_subcores=NS)
    return pl.kernel(
        _agg_body,
        out_type=jax.ShapeDtypeStruct((NC, N, H), jnp.float32),
        mesh=mesh,
        scratch_types=[
            pltpu.VMEM((CNT_F, CHUNK), jnp.int32),
            pltpu.VMEM((CNT_F, CHUNK), jnp.int32),
            pltpu.VMEM((CHUNK, H), jnp.float32),
            pltpu.VMEM((CHUNK, H), jnp.float32),
            pltpu.VMEM_SHARED((N + 128, H), jnp.float32),
            pltpu.VMEM_SHARED((N, H), jnp.float32),
            pltpu.SemaphoreType.DMA,
            pltpu.SemaphoreType.DMA,
        ],
        compiler_params=pltpu.CompilerParams(use_tc_tiling_on_sc=False),
    )


_sc_aggregate = _make_agg()


def _head_body(x_ref, w_ref, o_ref):
    o_ref[...] = jnp.dot(x_ref[...], w_ref[...],
                         preferred_element_type=jnp.float32)


def _head_matmul(x, w):
    d_in = x.shape[1]
    return pl.pallas_call(
        _head_body,
        grid=(GRID_N,),
        in_specs=[
            pl.BlockSpec((ROW_BLK, d_in), lambda i: (i, 0)),
            pl.BlockSpec((d_in, H), lambda i: (0, 0)),
        ],
        out_specs=pl.BlockSpec((ROW_BLK, H), lambda i: (i, 0)),
        out_shape=jax.ShapeDtypeStruct((N, H), jnp.float32),
    )(x, w)


def _tail_head_body(p0_ref, p1_ref, y_ref, b1_ref, g_ref, bt_ref, w2_ref,
                    b2_ref, w1n_ref, o_ref, *, use_bn):
    t = p0_ref[0] + p1_ref[0] - y_ref[...] + b1_ref[...]
    if use_bn:
        t = t * (_BN_SCALE * g_ref[...]) + bt_ref[...]
    t = _elu(t)
    t = jnp.dot(t, w2_ref[...], preferred_element_type=jnp.float32)
    z = _elu(t + b2_ref[...])
    o_ref[...] = jnp.dot(z, w1n_ref[...], preferred_element_type=jnp.float32)


def _tail_head(p, y, b1, gamma, beta, w2, b2, w1_next, use_bn):
    row = lambda v: v.reshape(1, H)
    vec_spec = pl.BlockSpec((1, H), lambda i: (0, 0))
    mat_spec = pl.BlockSpec((H, H), lambda i: (0, 0))
    blk_spec = pl.BlockSpec((ROW_BLK, H), lambda i: (i, 0))
    p_spec0 = pl.BlockSpec((1, ROW_BLK, H), lambda i: (0, i, 0))
    p_spec1 = pl.BlockSpec((1, ROW_BLK, H), lambda i: (1, i, 0))
    return pl.pallas_call(
        functools.partial(_tail_head_body, use_bn=use_bn),
        grid=(GRID_N,),
        in_specs=[p_spec0, p_spec1, blk_spec, vec_spec, vec_spec, vec_spec,
                  mat_spec, vec_spec, mat_spec],
        out_specs=blk_spec,
        out_shape=jax.ShapeDtypeStruct((N, H), jnp.float32),
    )(p, p, y, row(b1), row(gamma), row(beta), w2, row(b2), w1_next)


def _final_body(p0_ref, p1_ref, y_ref, b1_ref, w2_ref, b2_ref, batch_ref,
                mw1_ref, mb1_ref, mw2_ref, mb2_ref, mw3_ref, mb3_ref,
                o_ref, acc_ref):
    i = pl.program_id(0)
    t = p0_ref[0] + p1_ref[0] - y_ref[...] + b1_ref[...]
    t = _elu(t)
    t = jnp.dot(t, w2_ref[...], preferred_element_type=jnp.float32)
    z = _elu(t + b2_ref[...])
    b = batch_ref[0, 0, :]
    onehot_t = (lax.broadcasted_iota(jnp.int32, (G, ROW_BLK), 0)
                == b[None, :]).astype(jnp.float32)
    part = jnp.dot(onehot_t, z, preferred_element_type=jnp.float32)

    @pl.when(i == 0)
    def _():
        acc_ref[...] = jnp.zeros_like(acc_ref)

    acc_ref[...] += part

    @pl.when(i == GRID_N - 1)
    def _():
        pooled = acc_ref[...]
        r = _elu(jnp.dot(pooled, mw1_ref[...],
                         preferred_element_type=jnp.float32) + mb1_ref[...])
        r = _elu(jnp.dot(r, mw2_ref[...],
                         preferred_element_type=jnp.float32) + mb2_ref[...])
        o_ref[...] = jnp.dot(r, mw3_ref[...],
                             preferred_element_type=jnp.float32) + mb3_ref[...]


def _final(p, y, b1, w2, b2, batch3, mw1, mb1, mw2, mb2, mw3, mb3):
    vec = lambda v: v.reshape(1, -1)
    vec_spec = lambda n: pl.BlockSpec((1, n), lambda i: (0, 0))
    mat_spec = lambda m, n: pl.BlockSpec((m, n), lambda i: (0, 0))
    blk_spec = pl.BlockSpec((ROW_BLK, H), lambda i: (i, 0))
    return pl.pallas_call(
        _final_body,
        grid=(GRID_N,),
        in_specs=[
            pl.BlockSpec((1, ROW_BLK, H), lambda i: (0, i, 0)),
            pl.BlockSpec((1, ROW_BLK, H), lambda i: (1, i, 0)),
            blk_spec,
            vec_spec(H), mat_spec(H, H), vec_spec(H),
            pl.BlockSpec((1, 1, ROW_BLK), lambda i: (i, 0, 0)),
            mat_spec(H, H), vec_spec(H),
            mat_spec(H, H // 2), vec_spec(H // 2),
            mat_spec(H // 2, C), vec_spec(C),
        ],
        out_specs=pl.BlockSpec((G, C), lambda i: (0, 0)),
        out_shape=jax.ShapeDtypeStruct((G, C), jnp.float32),
        scratch_shapes=[pltpu.VMEM((G, H), jnp.float32)],
    )(p, p, y, vec(b1), w2, vec(b2), batch3, mw1, vec(mb1), mw2, vec(mb2),
      mw3, vec(mb3))


def kernel(x, edge_index, batch,
           pre0_w1, pre0_b1, pre0_gamma, pre0_beta, pre0_w2, pre0_b2,
           pre1_w1, pre1_b1, pre1_gamma, pre1_beta, pre1_w2, pre1_b2,
           post0_w1, post0_b1, post0_w2, post0_b2,
           mlp_w1, mlp_b1, mlp_w2, mlp_b2, mlp_w3, mlp_b3):
    pad_dst = N + (jnp.arange(E_PAD, dtype=jnp.int32) % 128)
    srcs = jnp.concatenate(
        [edge_index[0], jnp.zeros((E_PAD,), jnp.int32)]).reshape(TOTCH_PAD, CHUNK)
    dsts = jnp.concatenate([edge_index[1], pad_dst]).reshape(TOTCH_PAD, CHUNK)
    batch3 = batch.reshape(GRID_N, 1, ROW_BLK)

    y0 = _head_matmul(x, pre0_w1)
    p0 = _sc_aggregate(y0, srcs, dsts)
    y1 = _tail_head(p0, y0, pre0_b1, pre0_gamma, pre0_beta, pre0_w2, pre0_b2,
                    pre1_w1, use_bn=True)
    p1 = _sc_aggregate(y1, srcs, dsts)
    y2 = _tail_head(p1, y1, pre1_b1, pre1_gamma, pre1_beta, pre1_w2, pre1_b2,
                    post0_w1, use_bn=True)
    p2 = _sc_aggregate(y2, srcs, dsts)
    out = _final(p2, y2, post0_b1, post0_w2, post0_b2, batch3,
                 mlp_w1, mlp_b1, mlp_w2, mlp_b2, mlp_w3, mlp_b3)
    return (out, edge_index, batch)
